# baseline (speedup 1.0000x reference)
import functools

import jax
import jax.numpy as jnp
from jax import lax
from jax.experimental import pallas as pl
from jax.experimental.pallas import tpu as pltpu
from jax.experimental.pallas import tpu_sc as plsc

N = 10000
E = 160000
D = 256
DH = 128
NC = 2
NS = 16
LANES = 16
EPS = E // NS
CH = 80
CHA = 48
CHB = CH - CHA
NCHUNK = EPS // CH
ROWS_PS = 640
ROWS_LAST = N - 15 * ROWS_PS
PK = 25
BN_EPS = 1e-5
F32 = jnp.float32
BF16 = jnp.bfloat16
I32 = jnp.int32


def _sc_body(dst_h, src_h, a_h, x0_h, x1_h, out0_h, out1_h,
             dst_v, a_v, srcc_v, dstca_v, dstcb_v, denc_v, rowsb_v,
             f32a_v, f32b_v, zden_v,
             denom_sh, aggr_sh,
             gsem0, gsem1, dsem0, dsem1, ssema, ssemb, psem):
    cid = lax.axis_index("c")
    sid = lax.axis_index("s")
    ebase = sid * EPS
    rbase = sid * ROWS_PS

    pltpu.sync_copy(dst_h.at[pl.ds(ebase, EPS)], dst_v)
    pltpu.sync_copy(a_h.at[pl.ds(ebase, EPS)], a_v)

    zero16 = jnp.zeros((LANES,), F32)
    iota16 = lax.iota(I32, LANES)

    def zrow(i, c):
        row = f32b_v.at[i]
        for k in range(DH // LANES):
            row[pl.ds(k * LANES, LANES)] = zero16
        return c
    lax.fori_loop(0, CHB, zrow, 0)

    def zzd(i, c):
        zden_v[pl.ds(i * LANES, LANES)] = zero16
        return c
    lax.fori_loop(0, ROWS_PS // LANES, zzd, 0)

    @pl.when(sid < NS - 1)
    def _za_full():
        for t in range(ROWS_PS // CHB):
            pltpu.sync_copy(f32b_v, aggr_sh.at[pl.ds(rbase + t * CHB, CHB)])
        pltpu.sync_copy(zden_v, denom_sh.at[pl.ds(rbase, ROWS_PS)])

    @pl.when(sid == NS - 1)
    def _za_last():
        for t in range(ROWS_LAST // CHB):
            pltpu.sync_copy(f32b_v, aggr_sh.at[pl.ds(rbase + t * CHB, CHB)])
        pltpu.sync_copy(f32b_v.at[pl.ds(0, ROWS_LAST % CHB)],
                        aggr_sh.at[pl.ds(rbase + (ROWS_LAST // CHB) * CHB,
                                         ROWS_LAST % CHB)])
        pltpu.sync_copy(zden_v.at[pl.ds(0, ROWS_LAST)],
                        denom_sh.at[pl.ds(rbase, ROWS_LAST)])
    plsc.subcore_barrier()

    srcc = (srcc_v.at[0], srcc_v.at[1])
    dstca = (dstca_v.at[0], dstca_v.at[1])
    dstcb = (dstcb_v.at[0], dstcb_v.at[1])
    denc = (denc_v.at[0], denc_v.at[1])
    rowsb = (rowsb_v.at[0], rowsb_v.at[1])
    gsem = (gsem0, gsem1)
    dsem = (dsem0, dsem1)

    def _stage_src(c, b):
        pltpu.sync_copy(src_h.at[pl.ds(ebase + c * CH, CH)], srcc[b])

    def _start_gather(b):
        @pl.when(cid == 0)
        def _g0():
            pltpu.async_copy(x0_h.at[srcc[b]], rowsb[b], gsem[b])

        @pl.when(cid == 1)
        def _g1():
            pltpu.async_copy(x1_h.at[srcc[b]], rowsb[b], gsem[b])

    def _wait_gather(b):
        pltpu.make_async_copy(x0_h.at[srcc[b]], rowsb[b], gsem[b]).wait()

    def _start_den(c, b):
        pltpu.async_copy(denom_sh.at[dst_v.at[pl.ds(c * CH, CH)]],
                         denc[b], dsem[b])

    def _wait_den(b):
        pltpu.make_async_copy(denom_sh.at[dst_v.at[pl.ds(0, CH)]],
                              denc[b], dsem[b]).wait()

    def _wait_scatter_a(b):
        pltpu.make_async_copy(f32a_v, aggr_sh.at[dstca[b]], ssema).wait()

    def _wait_scatter_b(b):
        pltpu.make_async_copy(f32b_v, aggr_sh.at[dstcb[b]], ssemb).wait()

    _stage_src(0, 0)
    _start_gather(0)
    _stage_src(1, 1)
    _start_gather(1)

    def p1_round(r, c):
        def fire(i, c2):
            g = r * PK + i
            d16 = dst_v[pl.ds(g * LANES, LANES)]
            pltpu.async_copy(a_v.at[pl.ds(g * LANES, LANES)],
                             denom_sh.at[d16], psem, add=True)
            return c2
        lax.fori_loop(0, PK, fire, 0)

        def drain(i, c2):
            pltpu.make_async_copy(a_v.at[pl.ds(0, LANES)],
                                  denom_sh.at[iota16], psem).wait()
            return c2
        lax.fori_loop(0, PK, drain, 0)
        return c
    lax.fori_loop(0, EPS // LANES // PK, p1_round, 0)
    plsc.subcore_barrier()

    _start_den(0, 0)
    _start_den(1, 1)

    def _att(c, b):
        groups = []
        for j in range(CH // LANES):
            d16 = dst_v[pl.ds(c * CH + j * LANES, LANES)]
            a16 = a_v[pl.ds(c * CH + j * LANES, LANES)]
            den16 = denc[b][pl.ds(j * LANES, LANES)]
            groups.append(a16 / den16)
            if j < CHA // LANES:
                dstca[b][pl.ds(j * LANES, LANES)] = d16
            else:
                dstcb[b][pl.ds((j - CHA // LANES) * LANES, LANES)] = d16
        return groups

    def _scale_rows(b, groups, lo, hi, out_ref):
        for r in range(lo, hi):
            s = groups[r // LANES][r % LANES]
            row_i = rowsb_v.at[b, r]
            out = out_ref.at[r - lo]
            for k in range(DH // 32):
                v16 = row_i[pl.ds(k * LANES, LANES)]
                v32 = plsc.bitcast(v16, BF16)
                ua, ub = plsc.unpack(v32, format=plsc.PackFormat.INTERLEAVED)
                out[pl.ds(k * 32, LANES)] = ua * s
                out[pl.ds(k * 32 + LANES, LANES)] = ub * s

    def _process(c, b, guard_first):
        _wait_den(b)
        groups = _att(c, b)
        _wait_gather(b)

        def _half_a():
            _scale_rows(b, groups, 0, CHA, f32a_v)
            pltpu.async_copy(f32a_v, aggr_sh.at[dstca[b]], ssema, add=True)

        def _half_b():
            _scale_rows(b, groups, CHA, CH, f32b_v)
            pltpu.async_copy(f32b_v, aggr_sh.at[dstcb[b]], ssemb, add=True)

        if guard_first is None:
            _wait_scatter_a(b)
            _half_a()
            _wait_scatter_b(b)
            _half_b()
        else:
            @pl.when(guard_first)
            def _dr_a():
                _wait_scatter_a(b)
            _half_a()

            @pl.when(guard_first)
            def _dr_b():
                _wait_scatter_b(b)
            _half_b()

    def _restart(c, b):
        _stage_src(c, b)
        _start_gather(b)
        _start_den(c, b)

    def p2(i, carry):
        c0 = 2 * i
        _process(c0, 0, i > 0)
        _restart(c0 + 2, 0)

        _process(c0 + 1, 1, None)

        @pl.when(i < NCHUNK // 2 - 1)
        def _r1():
            _restart(c0 + 3, 1)
        return carry
    lax.fori_loop(0, NCHUNK // 2, p2, 0)

    _process(NCHUNK - 1, 0, None)
    _wait_scatter_a(0)
    _wait_scatter_b(0)
    plsc.subcore_barrier()

    @pl.when(sid < NS - 1)
    def _w_full():
        @pl.when(cid == 0)
        def _w0():
            pltpu.sync_copy(aggr_sh.at[pl.ds(rbase, ROWS_PS)],
                            out0_h.at[pl.ds(rbase, ROWS_PS)])

        @pl.when(cid == 1)
        def _w1():
            pltpu.sync_copy(aggr_sh.at[pl.ds(rbase, ROWS_PS)],
                            out1_h.at[pl.ds(rbase, ROWS_PS)])

    @pl.when(sid == NS - 1)
    def _w_last():
        @pl.when(cid == 0)
        def _w0():
            pltpu.sync_copy(aggr_sh.at[pl.ds(rbase, ROWS_LAST)],
                            out0_h.at[pl.ds(rbase, ROWS_LAST)])

        @pl.when(cid == 1)
        def _w1():
            pltpu.sync_copy(aggr_sh.at[pl.ds(rbase, ROWS_LAST)],
                            out1_h.at[pl.ds(rbase, ROWS_LAST)])


_sc_aggregate = pl.kernel(
    _sc_body,
    out_type=(jax.ShapeDtypeStruct((N, DH), F32),
              jax.ShapeDtypeStruct((N, DH), F32)),
    mesh=plsc.VectorSubcoreMesh(core_axis_name="c", subcore_axis_name="s",
                                num_cores=NC, num_subcores=NS),
    compiler_params=pltpu.CompilerParams(needs_layout_passes=False,
                                         use_tc_tiling_on_sc=False),
    scratch_types=[
        pltpu.VMEM((EPS,), I32),
        pltpu.VMEM((EPS,), F32),
        pltpu.VMEM((2, CH), I32),
        pltpu.VMEM((2, CHA), I32),
        pltpu.VMEM((2, CHB), I32),
        pltpu.VMEM((2, CH), F32),
        pltpu.VMEM((2, CH, DH // 2), I32),
        pltpu.VMEM((CHA, DH), F32),
        pltpu.VMEM((CHB, DH), F32),
        pltpu.VMEM((ROWS_PS,), F32),
        pltpu.VMEM_SHARED((N,), F32),
        pltpu.VMEM_SHARED((N, DH), F32),
        pltpu.SemaphoreType.DMA,
        pltpu.SemaphoreType.DMA,
        pltpu.SemaphoreType.DMA,
        pltpu.SemaphoreType.DMA,
        pltpu.SemaphoreType.DMA,
        pltpu.SemaphoreType.DMA,
        pltpu.SemaphoreType.DMA,
    ],
)


def _interleave_cols(y):
    n = y.shape[0]
    t = y.reshape(n, DH // 32, 2, LANES)
    t = jnp.swapaxes(t, 2, 3)
    return t.reshape(n, DH)


def _pack_rows(y):
    t = _interleave_cols(y).astype(BF16).reshape(y.shape[0], DH // 2, 2)
    return lax.bitcast_convert_type(t, I32)


def _tc_body(split, a0_ref, a1_ref, w_ref, b_ref, g_ref, bt_ref, *outs):
    w = w_ref[...]
    h = (jnp.dot(a0_ref[...], w[:DH, :], preferred_element_type=F32)
         + jnp.dot(a1_ref[...], w[DH:, :], preferred_element_type=F32)
         + b_ref[...])
    h = 0.5 * h * (1.0 + lax.erf(h * 0.7071067811865476))
    mean = jnp.mean(h, axis=0, keepdims=True)
    hc = h - mean
    var = jnp.mean(hc * hc, axis=0, keepdims=True)
    y = hc * lax.rsqrt(var + BN_EPS) * g_ref[...] + bt_ref[...]
    if split:
        outs[0][...] = y[:, :DH].astype(BF16)
        outs[1][...] = y[:, DH:].astype(BF16)
    else:
        outs[0][...] = y


_tc_update_split = pl.pallas_call(
    functools.partial(_tc_body, True),
    out_shape=(jax.ShapeDtypeStruct((N, DH), BF16),
               jax.ShapeDtypeStruct((N, DH), BF16)),
)

_tc_update_full = pl.pallas_call(
    functools.partial(_tc_body, False),
    out_shape=jax.ShapeDtypeStruct((N, D), F32),
)


def kernel(node_attr, edge_index, batch_idx, adv_atts,
           W0, b0, g0, bt0, W1, b1, g1, bt1):
    src = edge_index[0]
    dst = edge_index[1]
    x0 = _pack_rows(node_attr[:, :DH])
    x1 = _pack_rows(node_attr[:, DH:])
    ag0, ag1 = _sc_aggregate(dst, src, adv_atts[0], x0, x1)
    h0, h1 = _tc_update_split(ag0, ag1, W0, b0, g0, bt0)
    bg0, bg1 = _sc_aggregate(dst, src, adv_atts[1],
                             _pack_rows(h0), _pack_rows(h1))
    return _tc_update_full(bg0, bg1, W1, b1, g1, bt1)


# speedup vs baseline: 1.2889x; 1.2889x over previous
import functools

import jax
import jax.numpy as jnp
from jax import lax
from jax.experimental import pallas as pl
from jax.experimental.pallas import tpu as pltpu
from jax.experimental.pallas import tpu_sc as plsc

N = 10000
E = 160000
D = 256
DH = 128
NC = 2
NS = 16
LANES = 16
EPS = E // NS
CH = 80
NCHUNK = EPS // CH
ROWS_PS = 640
ROWS_LAST = N - 15 * ROWS_PS
PK = 25
BN_EPS = 1e-5
F32 = jnp.float32
I32 = jnp.int32


def _sc_body(dst_h, src_h, a_h, x0_h, x1_h, out0_h, out1_h,
             dst_v, a_v, srcc_v, dstc_v, denc_v, rows_v, zden_v,
             denom_sh, aggr_sh,
             gsem0, gsem1, ssem0, ssem1, dsem0, dsem1, csem0, csem1, psem):
    cid = lax.axis_index("c")
    sid = lax.axis_index("s")
    ebase = sid * EPS
    rbase = sid * ROWS_PS

    pltpu.sync_copy(dst_h.at[pl.ds(ebase, EPS)], dst_v)
    pltpu.sync_copy(a_h.at[pl.ds(ebase, EPS)], a_v)

    zero16 = jnp.zeros((LANES,), F32)

    def zrow(i, c):
        row = rows_v.at[0, i]
        for k in range(DH // LANES):
            row[pl.ds(k * LANES, LANES)] = zero16
        return c
    lax.fori_loop(0, CH, zrow, 0)

    def zzd(i, c):
        zden_v[pl.ds(i * LANES, LANES)] = zero16
        return c
    lax.fori_loop(0, ROWS_PS // LANES, zzd, 0)

    @pl.when(sid < NS - 1)
    def _za_full():
        for t in range(ROWS_PS // CH):
            pltpu.sync_copy(rows_v.at[0], aggr_sh.at[pl.ds(rbase + t * CH, CH)])
        pltpu.sync_copy(zden_v, denom_sh.at[pl.ds(rbase, ROWS_PS)])

    @pl.when(sid == NS - 1)
    def _za_last():
        for t in range(ROWS_LAST // CH):
            pltpu.sync_copy(rows_v.at[0], aggr_sh.at[pl.ds(rbase + t * CH, CH)])
        pltpu.sync_copy(zden_v.at[pl.ds(0, ROWS_LAST)],
                        denom_sh.at[pl.ds(rbase, ROWS_LAST)])
    plsc.subcore_barrier()

    srcc = (srcc_v.at[0], srcc_v.at[1])
    dstc = (dstc_v.at[0], dstc_v.at[1])
    denc = (denc_v.at[0], denc_v.at[1])
    rows = (rows_v.at[0], rows_v.at[1])
    gsem = (gsem0, gsem1)
    dsem = (dsem0, dsem1)
    ssem = (ssem0, ssem1)
    csem = (csem0, csem1)

    def _start_src(c, b):
        pltpu.async_copy(src_h.at[pl.ds(ebase + c * CH, CH)], srcc[b], csem[b])

    def _wait_src(b):
        pltpu.make_async_copy(src_h.at[pl.ds(ebase, CH)], srcc[b],
                              csem[b]).wait()

    def _start_gather(b):
        @pl.when(cid == 0)
        def _g0():
            pltpu.async_copy(x0_h.at[srcc[b]], rows[b], gsem[b])

        @pl.when(cid == 1)
        def _g1():
            pltpu.async_copy(x1_h.at[srcc[b]], rows[b], gsem[b])

    def _wait_gather(b):
        pltpu.make_async_copy(x0_h.at[srcc[b]], rows[b], gsem[b]).wait()

    def _start_den(c, b):
        pltpu.async_copy(denom_sh.at[dst_v.at[pl.ds(c * CH, CH)]],
                         denc[b], dsem[b])

    def _wait_den(b):
        pltpu.make_async_copy(denom_sh.at[dst_v.at[pl.ds(0, CH)]],
                              denc[b], dsem[b]).wait()

    def _start_scatter(b):
        pltpu.async_copy(rows[b], aggr_sh.at[dstc[b]], ssem[b], add=True)

    def _wait_scatter(b):
        pltpu.make_async_copy(rows[b], aggr_sh.at[dstc[b]], ssem[b]).wait()

    _start_src(0, 0)
    _wait_src(0)
    _start_gather(0)
    _start_src(1, 1)
    _wait_src(1)
    _start_gather(1)

    def p1_round(r, c):
        def fire(i, c2):
            g = r * PK + i
            pltpu.async_copy(a_v.at[pl.ds(g * CH, CH)],
                             denom_sh.at[dst_v.at[pl.ds(g * CH, CH)]],
                             psem, add=True)
            return c2
        lax.fori_loop(0, PK, fire, 0)

        def drain(i, c2):
            pltpu.make_async_copy(a_v.at[pl.ds(0, CH)],
                                  denom_sh.at[dst_v.at[pl.ds(0, CH)]],
                                  psem).wait()
            return c2
        lax.fori_loop(0, PK, drain, 0)
        return c
    lax.fori_loop(0, NCHUNK // PK, p1_round, 0)
    plsc.subcore_barrier()

    _start_den(0, 0)
    _start_den(1, 1)

    def _att(c, b):
        groups = []
        for j in range(CH // LANES):
            d16 = dst_v[pl.ds(c * CH + j * LANES, LANES)]
            a16 = a_v[pl.ds(c * CH + j * LANES, LANES)]
            den16 = denc[b][pl.ds(j * LANES, LANES)]
            groups.append(a16 / den16)
            dstc[b][pl.ds(j * LANES, LANES)] = d16
        return groups

    def _scale(b, groups):
        for g in range(CH // LANES):
            att16 = groups[g]
            for j in range(LANES):
                s = att16[j]
                row = rows_v.at[b, g * LANES + j]
                for k in range(DH // LANES):
                    sl = pl.ds(k * LANES, LANES)
                    row[sl] = row[sl] * s

    def _process(c, b, guard_first):
        _wait_den(b)
        groups = _att(c, b)
        _wait_gather(b)

        if guard_first is None:
            _wait_scatter(b)
        else:
            @pl.when(guard_first)
            def _dr():
                _wait_scatter(b)
        _scale(b, groups)
        _start_scatter(b)

    def _restart(c, b):
        _wait_src(b)
        _start_gather(b)
        _start_den(c, b)

    def p2(i, carry):
        c0 = 2 * i
        _wait_den(0)
        att0 = _att(c0, 0)
        _wait_gather(0)
        _start_src(c0 + 2, 0)

        @pl.when(i > 0)
        def _dr0():
            _wait_scatter(0)
        _scale(0, att0)
        _start_scatter(0)

        _restart(c0 + 2, 0)

        _wait_den(1)
        att1 = _att(c0 + 1, 1)
        _wait_gather(1)

        @pl.when(i < NCHUNK // 2 - 1)
        def _s1():
            _start_src(c0 + 3, 1)

        @pl.when(i > 0)
        def _dr1():
            _wait_scatter(1)
        _scale(1, att1)
        _start_scatter(1)

        @pl.when(i < NCHUNK // 2 - 1)
        def _r1():
            _restart(c0 + 3, 1)
        return carry
    lax.fori_loop(0, NCHUNK // 2, p2, 0)

    _wait_den(0)
    attL = _att(NCHUNK - 1, 0)
    _wait_gather(0)
    _wait_scatter(0)
    _scale(0, attL)
    _start_scatter(0)
    _wait_scatter(0)
    _wait_scatter(1)
    plsc.subcore_barrier()

    @pl.when(sid < NS - 1)
    def _w_full():
        @pl.when(cid == 0)
        def _w0():
            pltpu.sync_copy(aggr_sh.at[pl.ds(rbase, ROWS_PS)],
                            out0_h.at[pl.ds(rbase, ROWS_PS)])

        @pl.when(cid == 1)
        def _w1():
            pltpu.sync_copy(aggr_sh.at[pl.ds(rbase, ROWS_PS)],
                            out1_h.at[pl.ds(rbase, ROWS_PS)])

    @pl.when(sid == NS - 1)
    def _w_last():
        @pl.when(cid == 0)
        def _w0():
            pltpu.sync_copy(aggr_sh.at[pl.ds(rbase, ROWS_LAST)],
                            out0_h.at[pl.ds(rbase, ROWS_LAST)])

        @pl.when(cid == 1)
        def _w1():
            pltpu.sync_copy(aggr_sh.at[pl.ds(rbase, ROWS_LAST)],
                            out1_h.at[pl.ds(rbase, ROWS_LAST)])


_sc_aggregate = pl.kernel(
    _sc_body,
    out_type=(jax.ShapeDtypeStruct((N, DH), F32),
              jax.ShapeDtypeStruct((N, DH), F32)),
    mesh=plsc.VectorSubcoreMesh(core_axis_name="c", subcore_axis_name="s",
                                num_cores=NC, num_subcores=NS),
    compiler_params=pltpu.CompilerParams(needs_layout_passes=False),
    scratch_types=[
        pltpu.VMEM((EPS,), I32),
        pltpu.VMEM((EPS,), F32),
        pltpu.VMEM((2, CH), I32),
        pltpu.VMEM((2, CH), I32),
        pltpu.VMEM((2, CH), F32),
        pltpu.VMEM((2, CH, DH), F32),
        pltpu.VMEM((ROWS_PS,), F32),
        pltpu.VMEM_SHARED((N,), F32),
        pltpu.VMEM_SHARED((N, DH), F32),
        pltpu.SemaphoreType.DMA,
        pltpu.SemaphoreType.DMA,
        pltpu.SemaphoreType.DMA,
        pltpu.SemaphoreType.DMA,
        pltpu.SemaphoreType.DMA,
        pltpu.SemaphoreType.DMA,
        pltpu.SemaphoreType.DMA,
        pltpu.SemaphoreType.DMA,
        pltpu.SemaphoreType.DMA,
    ],
)


def _tc_body(split, a0_ref, a1_ref, w_ref, b_ref, g_ref, bt_ref, *outs):
    w = w_ref[...]
    h = (jnp.dot(a0_ref[...], w[:DH, :], preferred_element_type=F32)
         + jnp.dot(a1_ref[...], w[DH:, :], preferred_element_type=F32)
         + b_ref[...])
    h = 0.5 * h * (1.0 + lax.erf(h * 0.7071067811865476))
    mean = jnp.mean(h, axis=0, keepdims=True)
    hc = h - mean
    var = jnp.mean(hc * hc, axis=0, keepdims=True)
    y = hc * lax.rsqrt(var + BN_EPS) * g_ref[...] + bt_ref[...]
    if split:
        outs[0][...] = y[:, :DH]
        outs[1][...] = y[:, DH:]
    else:
        outs[0][...] = y


_tc_update_split = pl.pallas_call(
    functools.partial(_tc_body, True),
    out_shape=(jax.ShapeDtypeStruct((N, DH), F32),
               jax.ShapeDtypeStruct((N, DH), F32)),
)

_tc_update_full = pl.pallas_call(
    functools.partial(_tc_body, False),
    out_shape=jax.ShapeDtypeStruct((N, D), F32),
)


def kernel(node_attr, edge_index, batch_idx, adv_atts,
           W0, b0, g0, bt0, W1, b1, g1, bt1):
    src = edge_index[0]
    dst = edge_index[1]
    x0 = node_attr[:, :DH]
    x1 = node_attr[:, DH:]
    ag0, ag1 = _sc_aggregate(dst, src, adv_atts[0], x0, x1)
    h0, h1 = _tc_update_split(ag0, ag1, W0, b0, g0, bt0)
    bg0, bg1 = _sc_aggregate(dst, src, adv_atts[1], h0, h1)
    return _tc_update_full(bg0, bg1, W1, b1, g1, bt1)
